# baseline (speedup 1.0000x reference)
import functools

import jax
import jax.numpy as jnp
from jax import lax
from jax.experimental import pallas as pl
from jax.experimental.pallas import tpu as pltpu
from jax.experimental.pallas import tpu_sc as plsc

NBLK = 2048
B = 8
C = 512
D = 3
N = 4096
KK = B // 2
NEG = float(-3.0e38)
IMIN = -2147483647


def _score_body(fx_ref, fy_ref, s_ref):
    for sub in range(NBLK // 128):
        lo = sub * 128
        fxb = fx_ref[0, :, :, lo:lo + 128]
        fyb = fy_ref[0, :, :, lo:lo + 128]
        acc_m = jnp.zeros((3, 3, 8, 128), jnp.float32)
        acc_x = jnp.zeros((3, 8, 128), jnp.float32)
        acc_y = jnp.zeros((3, 8, 128), jnp.float32)
        for a in range(C // 8):
            xa = fxb[:, 8 * a:8 * a + 8, :]
            ya = fyb[:, 8 * a:8 * a + 8, :]
            acc_m = acc_m + xa[:, None] * ya[None, :]
            acc_x = acc_x + xa
            acc_y = acc_y + ya
        m = jnp.sum(acc_m, axis=2)
        sx = jnp.sum(acc_x, axis=1)
        sy = jnp.sum(acc_y, axis=1)
        inv_c = jnp.float32(1.0 / C)
        mx = sx * inv_c
        my = sy * inv_c
        nx = jnp.sqrt(jnp.sum(mx * mx, axis=0)) + jnp.float32(1e-6)
        ny = jnp.sqrt(jnp.sum(my * my, axis=0)) + jnp.float32(1e-6)
        ux = mx / nx
        uy = my / ny
        s = jnp.sum(ux[:, None, :] * uy[None, :, :] * m, axis=(0, 1))
        s_ref[0, 0, lo:lo + 128] = s


def _scores(fxt, fyt):
    return pl.pallas_call(
        _score_body,
        grid=(B, N // NBLK),
        in_specs=[
            pl.BlockSpec((1, D, C, NBLK), lambda b, nb: (b, 0, 0, nb)),
            pl.BlockSpec((1, D, C, NBLK), lambda b, nb: (b, 0, 0, nb)),
        ],
        out_specs=pl.BlockSpec((1, 1, NBLK), lambda b, nb: (b, 0, nb)),
        out_shape=jax.ShapeDtypeStruct((B, 1, N), jnp.float32),
        compiler_params=pltpu.CompilerParams(
            dimension_semantics=("arbitrary", "arbitrary"),
            vmem_limit_bytes=60 * 1024 * 1024),
    )(fxt, fyt)


def _sc_topk(s):
    mesh = plsc.VectorSubcoreMesh(core_axis_name="c", subcore_axis_name="s")
    chunk_len = N // 4

    @functools.partial(
        pl.kernel,
        mesh=mesh,
        out_type=jax.ShapeDtypeStruct((B, 16), jnp.int32),
        scratch_types=[
            pltpu.VMEM((chunk_len,), jnp.float32),
            pltpu.VMEM((16,), jnp.float32),
            pltpu.VMEM((16,), jnp.int32),
            pltpu.VMEM((4, 16), jnp.float32),
            pltpu.VMEM((4, 16), jnp.int32),
            pltpu.VMEM_SHARED((4, 4, 16), jnp.float32),
            pltpu.VMEM_SHARED((4, 4, 16), jnp.int32),
        ],
        compiler_params=pltpu.CompilerParams(
            use_tc_tiling_on_sc=False, needs_layout_passes=False),
    )
    def k(s_hbm, oi_hbm,
          sv, cand_v, cand_i, merge_v, merge_i, shv, shi):
        ci = lax.axis_index("c")
        si = lax.axis_index("s")
        bl = si // 4
        chunk = si % 4
        batch = ci * 4 + bl
        base = chunk * chunk_len
        pltpu.sync_copy(s_hbm.at[batch, pl.ds(base, chunk_len)], sv)
        lane = lax.iota(jnp.int32, 16)

        def top4(load_row, nrows, fori):
            winners_v, winners_i = [], []
            for _ in range(4):
                prev = list(winners_i)

                def body(j, carry, prev=prev):
                    mv, mi = carry
                    v, iv = load_row(j)
                    for w in prev:
                        v = jnp.where(iv == w, NEG, v)
                    gt = v > mv
                    return jnp.where(gt, v, mv), jnp.where(gt, iv, mi)

                mv = jnp.full((16,), NEG, jnp.float32)
                mi = jnp.zeros((16,), jnp.int32)
                if fori:
                    mv, mi = lax.fori_loop(0, nrows, body, (mv, mi))
                else:
                    for j in range(nrows):
                        mv, mi = body(j, (mv, mi))
                best = jnp.max(mv)
                bi = -jnp.max(jnp.where(mv == best, -mi, jnp.int32(IMIN)))
                winners_v.append(best)
                winners_i.append(bi)
            return winners_v, winners_i

        def load_chunk(j):
            v = sv[pl.ds(j * 16, 16)]
            iv = base + j * 16 + lane
            return v, iv

        wv, wi = top4(load_chunk, chunk_len // 16, fori=True)
        cv = jnp.full((16,), NEG, jnp.float32)
        civ = jnp.zeros((16,), jnp.int32)
        for k4 in range(4):
            cv = jnp.where(lane == k4, wv[k4], cv)
            civ = jnp.where(lane == k4, wi[k4], civ)
        cand_v[...] = cv
        cand_i[...] = civ
        pltpu.sync_copy(cand_v, shv.at[bl, chunk])
        pltpu.sync_copy(cand_i, shi.at[bl, chunk])
        plsc.subcore_barrier()

        @pl.when(chunk == 0)
        def _():
            pltpu.sync_copy(shv.at[bl], merge_v)
            pltpu.sync_copy(shi.at[bl], merge_i)

            def load_merge(j):
                return merge_v[j], merge_i[j]

            _, fin = top4(load_merge, 4, fori=False)
            fiv = jnp.zeros((16,), jnp.int32)
            for k4 in range(KK):
                fiv = jnp.where(lane == k4, fin[k4], fiv)
            cand_i[...] = fiv
            pltpu.sync_copy(cand_i, oi_hbm.at[batch])

    return k(s)


def _gather_body(idx_ref, *refs):
    in_refs = refs[:2 * KK]
    ox_ref, oy_ref = refs[2 * KK:]
    bb = pl.program_id(0)
    lanei = lax.broadcasted_iota(jnp.int32, (1, 1, 128), 2)
    for k4 in range(KK):
        col = lax.rem(idx_ref[bb, k4], 128)
        sel = lanei == col
        ox_ref[0, k4] = jnp.sum(jnp.where(sel, in_refs[k4][0], 0.0), axis=2)
        oy_ref[0, k4] = jnp.sum(jnp.where(sel, in_refs[KK + k4][0], 0.0),
                                axis=2)


def _gather(idx, fxt, fyt):
    def slab_spec(k4):
        return pl.BlockSpec((1, D, C, 128),
                            lambda b, idx, k4=k4: (b, 0, 0, idx[b, k4] // 128))

    grid_spec = pltpu.PrefetchScalarGridSpec(
        num_scalar_prefetch=1,
        grid=(B,),
        in_specs=[slab_spec(k4) for k4 in range(KK)] * 2,
        out_specs=[
            pl.BlockSpec((1, KK, D, C), lambda b, idx: (b, 0, 0, 0)),
            pl.BlockSpec((1, KK, D, C), lambda b, idx: (b, 0, 0, 0)),
        ],
    )
    return pl.pallas_call(
        _gather_body,
        grid_spec=grid_spec,
        out_shape=[jax.ShapeDtypeStruct((B, KK, D, C), jnp.float32),
                   jax.ShapeDtypeStruct((B, KK, D, C), jnp.float32)],
    )(idx, *([fxt] * KK), *([fyt] * KK))


def kernel(fx, fy, topk):
    b, c, d, n = fx.shape
    assert (b, c, d, n) == (B, C, D, N), (b, c, d, n)
    fxt = fx.transpose(0, 2, 1, 3)
    fyt = fy.transpose(0, 2, 1, 3)
    s = _scores(fxt, fyt).reshape(b, n)
    idx = _sc_topk(s)[:, :KK]
    gx, gy = _gather(idx, fxt, fyt)
    return gx.transpose(0, 3, 2, 1), gy.transpose(0, 3, 2, 1)


# speedup vs baseline: 1.0150x; 1.0150x over previous
import functools

import jax
import jax.numpy as jnp
from jax import lax
from jax.experimental import pallas as pl
from jax.experimental.pallas import tpu as pltpu
from jax.experimental.pallas import tpu_sc as plsc

NBLK = 1024
B = 8
C = 512
D = 3
N = 4096
KK = B // 2
NEG = float(-3.0e38)
IMIN = -2147483647


def _score_body(fx_ref, fy_ref, s_ref):
    for sub in range(NBLK // 128):
        lo = sub * 128
        fxb = fx_ref[0, :, :, lo:lo + 128]
        fyb = fy_ref[0, :, :, lo:lo + 128]
        acc_m = jnp.zeros((3, 3, 8, 128), jnp.float32)
        acc_x = jnp.zeros((3, 8, 128), jnp.float32)
        acc_y = jnp.zeros((3, 8, 128), jnp.float32)
        for a in range(C // 8):
            xa = fxb[:, 8 * a:8 * a + 8, :]
            ya = fyb[:, 8 * a:8 * a + 8, :]
            acc_m = acc_m + xa[:, None] * ya[None, :]
            acc_x = acc_x + xa
            acc_y = acc_y + ya
        m = jnp.sum(acc_m, axis=2)
        sx = jnp.sum(acc_x, axis=1)
        sy = jnp.sum(acc_y, axis=1)
        inv_c = jnp.float32(1.0 / C)
        mx = sx * inv_c
        my = sy * inv_c
        nx = jnp.sqrt(jnp.sum(mx * mx, axis=0)) + jnp.float32(1e-6)
        ny = jnp.sqrt(jnp.sum(my * my, axis=0)) + jnp.float32(1e-6)
        ux = mx / nx
        uy = my / ny
        s = jnp.sum(ux[:, None, :] * uy[None, :, :] * m, axis=(0, 1))
        s_ref[0, 0, lo:lo + 128] = s


def _scores(fxt, fyt):
    return pl.pallas_call(
        _score_body,
        grid=(B, N // NBLK),
        in_specs=[
            pl.BlockSpec((1, D, C, NBLK), lambda b, nb: (b, 0, 0, nb)),
            pl.BlockSpec((1, D, C, NBLK), lambda b, nb: (b, 0, 0, nb)),
        ],
        out_specs=pl.BlockSpec((1, 1, NBLK), lambda b, nb: (b, 0, nb)),
        out_shape=jax.ShapeDtypeStruct((B, 1, N), jnp.float32),
        compiler_params=pltpu.CompilerParams(
            dimension_semantics=("arbitrary", "arbitrary")),
    )(fxt, fyt)


def _sc_topk(s):
    mesh = plsc.VectorSubcoreMesh(core_axis_name="c", subcore_axis_name="s")
    chunk_len = N // 4

    @functools.partial(
        pl.kernel,
        mesh=mesh,
        out_type=jax.ShapeDtypeStruct((B, 16), jnp.int32),
        scratch_types=[
            pltpu.VMEM((chunk_len,), jnp.float32),
            pltpu.VMEM((16,), jnp.float32),
            pltpu.VMEM((16,), jnp.int32),
            pltpu.VMEM((4, 16), jnp.float32),
            pltpu.VMEM((4, 16), jnp.int32),
            pltpu.VMEM_SHARED((4, 4, 16), jnp.float32),
            pltpu.VMEM_SHARED((4, 4, 16), jnp.int32),
        ],
        compiler_params=pltpu.CompilerParams(
            use_tc_tiling_on_sc=False, needs_layout_passes=False),
    )
    def k(s_hbm, oi_hbm,
          sv, cand_v, cand_i, merge_v, merge_i, shv, shi):
        ci = lax.axis_index("c")
        si = lax.axis_index("s")
        bl = si // 4
        chunk = si % 4
        batch = ci * 4 + bl
        base = chunk * chunk_len
        pltpu.sync_copy(s_hbm.at[batch, pl.ds(base, chunk_len)], sv)
        lane = lax.iota(jnp.int32, 16)

        def top4(load_row, nrows, fori):
            winners_v, winners_i = [], []
            for _ in range(4):
                prev = list(winners_i)

                def body(j, carry, prev=prev):
                    mv, mi = carry
                    v, iv = load_row(j)
                    for w in prev:
                        v = jnp.where(iv == w, NEG, v)
                    gt = v > mv
                    return jnp.where(gt, v, mv), jnp.where(gt, iv, mi)

                mv = jnp.full((16,), NEG, jnp.float32)
                mi = jnp.zeros((16,), jnp.int32)
                if fori:
                    mv, mi = lax.fori_loop(0, nrows, body, (mv, mi))
                else:
                    for j in range(nrows):
                        mv, mi = body(j, (mv, mi))
                best = jnp.max(mv)
                bi = -jnp.max(jnp.where(mv == best, -mi, jnp.int32(IMIN)))
                winners_v.append(best)
                winners_i.append(bi)
            return winners_v, winners_i

        def load_chunk(j):
            v = sv[pl.ds(j * 16, 16)]
            iv = base + j * 16 + lane
            return v, iv

        wv, wi = top4(load_chunk, chunk_len // 16, fori=True)
        cv = jnp.full((16,), NEG, jnp.float32)
        civ = jnp.zeros((16,), jnp.int32)
        for k4 in range(4):
            cv = jnp.where(lane == k4, wv[k4], cv)
            civ = jnp.where(lane == k4, wi[k4], civ)
        cand_v[...] = cv
        cand_i[...] = civ
        pltpu.sync_copy(cand_v, shv.at[bl, chunk])
        pltpu.sync_copy(cand_i, shi.at[bl, chunk])
        plsc.subcore_barrier()

        @pl.when(chunk == 0)
        def _():
            pltpu.sync_copy(shv.at[bl], merge_v)
            pltpu.sync_copy(shi.at[bl], merge_i)

            def load_merge(j):
                return merge_v[j], merge_i[j]

            _, fin = top4(load_merge, 4, fori=False)
            fiv = jnp.zeros((16,), jnp.int32)
            for k4 in range(KK):
                fiv = jnp.where(lane == k4, fin[k4], fiv)
            cand_i[...] = fiv
            pltpu.sync_copy(cand_i, oi_hbm.at[batch])

    return k(s)


def _gather_body(idx_ref, *refs):
    in_refs = refs[:2 * KK]
    ox_ref, oy_ref = refs[2 * KK:]
    bb = pl.program_id(0)
    lanei = lax.broadcasted_iota(jnp.int32, (1, 1, 128), 2)
    for k4 in range(KK):
        col = lax.rem(idx_ref[bb, k4], 128)
        sel = lanei == col
        ox_ref[0, k4] = jnp.sum(jnp.where(sel, in_refs[k4][0], 0.0), axis=2)
        oy_ref[0, k4] = jnp.sum(jnp.where(sel, in_refs[KK + k4][0], 0.0),
                                axis=2)


def _gather(idx, fxt, fyt):
    def slab_spec(k4):
        return pl.BlockSpec((1, D, C, 128),
                            lambda b, idx, k4=k4: (b, 0, 0, idx[b, k4] // 128))

    grid_spec = pltpu.PrefetchScalarGridSpec(
        num_scalar_prefetch=1,
        grid=(B,),
        in_specs=[slab_spec(k4) for k4 in range(KK)] * 2,
        out_specs=[
            pl.BlockSpec((1, KK, D, C), lambda b, idx: (b, 0, 0, 0)),
            pl.BlockSpec((1, KK, D, C), lambda b, idx: (b, 0, 0, 0)),
        ],
    )
    return pl.pallas_call(
        _gather_body,
        grid_spec=grid_spec,
        out_shape=[jax.ShapeDtypeStruct((B, KK, D, C), jnp.float32),
                   jax.ShapeDtypeStruct((B, KK, D, C), jnp.float32)],
    )(idx, *([fxt] * KK), *([fyt] * KK))


def kernel(fx, fy, topk):
    b, c, d, n = fx.shape
    assert (b, c, d, n) == (B, C, D, N), (b, c, d, n)
    fxt = fx.transpose(0, 2, 1, 3)
    fyt = fy.transpose(0, 2, 1, 3)
    s = _scores(fxt, fyt).reshape(b, n)
    idx = _sc_topk(s)[:, :KK]
    gx, gy = _gather(idx, fxt, fyt)
    return gx.transpose(0, 3, 2, 1), gy.transpose(0, 3, 2, 1)
